# baseline (speedup 1.0000x reference)
import functools
import jax
import jax.numpy as jnp
from jax import lax
from jax.experimental import pallas as pl
from jax.experimental.pallas import tpu as pltpu
from jax.experimental.pallas import tpu_sc as plsc

CLS = 64
CH = 128
WT = 512
GB = 16
T = 128
NW = 32
LANE = 16


def _lrelu(x):
    return jnp.where(x >= 0, x, 0.01 * x)


def _vgather(vec, idx):
    dn = lax.GatherDimensionNumbers(
        offset_dims=(), collapsed_slice_dims=(0,), start_index_map=(0,))
    return lax.gather(vec, idx[:, None], dn, (1,),
                      mode=lax.GatherScatterMode.PROMISE_IN_BOUNDS)


def _fused_body(
    x_ref, wcl1_ref, g1_ref, be1_ref, wcl2_ref, bcl2_ref, wcl3_ref, bcl3_ref,
    wrg_ref, gr_ref, br_ref, inv_n_ref,
    mask_ref, inds_ref, xr_ref, hist_ref, acc,
):
    p = pl.program_id(0)
    step = pl.program_id(1) * pl.num_programs(2) + pl.program_id(2)
    x = x_ref[0, :, 0, :]
    wt = x.shape[1]

    @pl.when(p == 0)
    def _():
        ycl = jnp.dot(wcl1_ref[...], x, preferred_element_type=jnp.float32)
        yrg = jnp.dot(wrg_ref[...], x, preferred_element_type=jnp.float32)
        snew = jnp.concatenate(
            [
                jnp.sum(ycl, axis=1, keepdims=True),
                jnp.sum(ycl * ycl, axis=1, keepdims=True),
                jnp.sum(yrg, axis=1, keepdims=True),
                jnp.sum(yrg * yrg, axis=1, keepdims=True),
            ],
            axis=1,
        )
        acc[...] = jnp.where(step == 0, snew, acc[...] + snew)

    @pl.when(p == 1)
    def _():
        eps = 1e-5
        inv_n = inv_n_ref[0, 0]
        st = acc[...] * inv_n
        m_cl, q_cl = st[:, 0:1], st[:, 1:2]
        m_rg, q_rg = st[:, 2:3], st[:, 3:4]
        sc1 = g1_ref[...] / jnp.sqrt(q_cl - m_cl * m_cl + eps)
        sh1 = be1_ref[...] - m_cl * sc1
        scr = gr_ref[...] / jnp.sqrt(q_rg - m_rg * m_rg + eps)
        shr = br_ref[...] - m_rg * scr

        y = jnp.dot(wcl1_ref[...], x, preferred_element_type=jnp.float32)
        h1 = _lrelu(y * sc1 + sh1)
        h2 = _lrelu(jnp.dot(wcl2_ref[...], h1, preferred_element_type=jnp.float32)
                    + bcl2_ref[...][:, 0:1])
        logits = (jnp.dot(wcl3_ref[...], h2, preferred_element_type=jnp.float32)
                  + bcl3_ref[...][:, 0:1])
        cls = logits[0:CLS, :]
        m = jnp.max(cls, axis=0, keepdims=True)
        iota = lax.broadcasted_iota(jnp.int32, (CLS, wt), 0)
        ind = jnp.min(jnp.where(cls == m, iota, CLS), axis=0, keepdims=True)
        mask_ref[0, 0] = _lrelu(logits[CLS : CLS + 1, :])
        inds_ref[0, 0] = ind

        onehot = (iota == ind).astype(jnp.int32)
        hsum = jnp.sum(onehot, axis=1, keepdims=True)
        hist_ref[0] = jnp.transpose(hsum)

        yr = jnp.dot(wrg_ref[...], x, preferred_element_type=jnp.float32)
        xr = _lrelu(yr * scr + shr)
        xr_ref[...] = jnp.transpose(xr)


def _rank_in_vreg(idx, lane):
    sk, sv = plsc.sort_key_val(idx, lane)
    prev = _vgather(sk, jnp.maximum(lane - 1, 0))
    change = (lane == 0) | (sk != prev)
    start = plsc.cummax(jnp.where(change, lane, 0))
    rank = lane - start
    nxt = _vgather(sk, jnp.minimum(lane + 1, 15))
    is_last = (lane == 15) | (sk != nxt)
    return sk, sv, rank, is_last


def _make_route_kernel(N, NTILES, NP, NPT, GIDPAD, mesh):
    chunk = N // NW
    nv = chunk // LANE

    @functools.partial(
        pl.kernel, mesh=mesh,
        out_type=[
            jax.ShapeDtypeStruct((NW, chunk // T, T), jnp.int32),
            jax.ShapeDtypeStruct((NP, CH), jnp.float32),
            jax.ShapeDtypeStruct((NPT,), jnp.int32),
        ],
        scratch_types=[
            pltpu.VMEM((chunk,), jnp.int32),
            pltpu.VMEM((NTILES, CLS), jnp.int32),
            pltpu.VMEM((CLS,), jnp.int32),
            pltpu.VMEM((chunk // T, T), jnp.int32),
            pltpu.VMEM((LANE,), jnp.int32),
            pltpu.VMEM((GIDPAD,), jnp.int32),
            pltpu.VMEM((2, T * 2, CH), jnp.float32),
            pltpu.SemaphoreType.DMA,
            pltpu.SemaphoreType.DMA,
            pltpu.SemaphoreType.DMA,
            pltpu.SemaphoreType.DMA,
        ],
        compiler_params=pltpu.CompilerParams(needs_layout_passes=False),
    )
    def route(inds_hbm, hist_hbm, xr_hbm, dest_hbm, xs_hbm, gid_hbm,
              idx_v, hist_v, cnt_v, dest2d_v, tmp16_v, gid_v, rows_v,
              semc0, semc1, sems0, sems1):
        wid = lax.axis_index("s") * 2 + lax.axis_index("c")
        lane = lax.iota(jnp.int32, LANE)
        pltpu.sync_copy(hist_hbm, hist_v)
        pltpu.sync_copy(inds_hbm.at[pl.ds(wid * chunk, chunk)], idx_v)

        tiles_per_chunk = NTILES // NW
        tot, par = [], []
        for j in range(4):
            acc = jnp.zeros((LANE,), jnp.int32)
            pacc = jnp.zeros((LANE,), jnp.int32)
            for t in range(NTILES):
                row = hist_v[t, pl.ds(j * LANE, LANE)]
                acc = acc + row
                pred = (t < wid * tiles_per_chunk).astype(jnp.int32)
                pacc = pacc + row * pred
            tot.append(acc)
            par.append(pacc)

        carry = jnp.int32(0)
        tile_off = []
        for j in range(4):
            aligned = ((tot[j] + (T - 1)) >> 7) << 7
            cs = plsc.cumsum(aligned)
            off = cs - aligned + carry
            carry = carry + jnp.sum(aligned)
            cnt_v[pl.ds(j * LANE, LANE)] = off + par[j]
            tile_off.append(off >> 7)

        @pl.when(wid == 0)
        def _():
            for jv in range(GIDPAD // LANE):
                gid_v[pl.ds(jv * LANE, LANE)] = jnp.zeros((LANE,), jnp.int32)
            for j in range(4):
                sk, sv, rank, is_last = _rank_in_vreg(tile_off[j], lane)
                base = plsc.load_gather(gid_v, [sk])
                plsc.store_scatter(gid_v, [sk], base + rank + 1, mask=is_last)
            c2 = jnp.int32(0)
            for jv in range(GIDPAD // LANE):
                seg = gid_v[pl.ds(jv * LANE, LANE)]
                s = jnp.sum(seg)
                gid_v[pl.ds(jv * LANE, LANE)] = plsc.cumsum(seg) + c2 - 1
                c2 = c2 + s
            pltpu.sync_copy(gid_v.at[pl.ds(0, NPT)], gid_hbm)

        for v in range(nv):
            idx = idx_v[pl.ds(v * LANE, LANE)]
            sk, sv, rank, is_last = _rank_in_vreg(idx, lane)
            base = plsc.load_gather(cnt_v, [sk])
            plsc.store_scatter(cnt_v, [sk], base + rank + 1, mask=is_last)
            plsc.store_scatter(tmp16_v, [sv], base + rank)
            dest2d_v[v // 8, pl.ds((v % 8) * LANE, LANE)] = tmp16_v[...]

        pltpu.sync_copy(dest2d_v, dest_hbm.at[wid])

        nsub = chunk // (T * 2)
        semc = [semc0, semc1]
        sems = [sems0, sems1]

        def _start_copy(sub):
            buf = sub % 2
            return pltpu.async_copy(
                xr_hbm.at[pl.ds(wid * chunk + sub * T * 2, T * 2)],
                rows_v.at[buf], semc[buf])

        copy_descs = {0: _start_copy(0)}
        scat_descs = {}
        for sub in range(nsub):
            buf = sub % 2
            copy_descs[sub].wait()
            scat_descs[sub] = [
                pltpu.async_copy(
                    rows_v.at[buf, pl.ds(j * T, T)],
                    xs_hbm.at[dest2d_v.at[sub * 2 + j]],
                    sems[buf],
                )
                for j in range(2)
            ]
            if sub + 1 < nsub:
                if sub >= 1:
                    for d in scat_descs[sub - 1]:
                        d.wait()
                copy_descs[sub + 1] = _start_copy(sub + 1)
        for sub in (nsub - 2, nsub - 1):
            if sub >= 0 and sub in scat_descs:
                for d in scat_descs[sub]:
                    d.wait()

    return route


def _make_gather_kernel(N, NP, mesh):
    chunk = N // NW
    nv = chunk // LANE

    @functools.partial(
        pl.kernel, mesh=mesh,
        out_type=jax.ShapeDtypeStruct((NW, chunk // T, T), jnp.float32),
        scratch_types=[
            pltpu.VMEM((NP,), jnp.float32),
            pltpu.VMEM((chunk // T, T), jnp.int32),
            pltpu.VMEM((chunk // T, T), jnp.float32),
        ],
        compiler_params=pltpu.CompilerParams(needs_layout_passes=False),
    )
    def gather_back(val_hbm, dest_hbm, out_hbm, val_v, dv, ov):
        wid = lax.axis_index("s") * 2 + lax.axis_index("c")
        pltpu.sync_copy(val_hbm, val_v)
        pltpu.sync_copy(dest_hbm.at[wid], dv)
        for v in range(nv):
            d = dv[v // 8, pl.ds((v % 8) * LANE, LANE)]
            ov[v // 8, pl.ds((v % 8) * LANE, LANE)] = plsc.load_gather(val_v, [d])
        pltpu.sync_copy(ov, out_hbm.at[wid])

    return gather_back


def _group_body(gid_ref, xs_ref, *refs):
    pk_refs = refs[0:GB]
    out_ref = refs[GB]
    i = pl.program_id(0)
    svals = []
    offs = []
    for k in range(GB):
        g = gid_ref[i * GB + k]
        x = xs_ref[pl.ds(k * T, T), :].astype(jnp.bfloat16)
        pk = pk_refs[k][0]
        z = jnp.dot(x, pk[0:CH, :], preferred_element_type=jnp.float32)
        b2 = pk[CH:CH + 1, :].astype(jnp.float32)
        w3 = pk[CH + 1:CH + 2, :].astype(jnp.float32)
        b3 = pk[CH + 2:CH + 3, 0:1].astype(jnp.float32)
        svals.append(_lrelu(z + b2) * w3)
        offs.append(jnp.full((1, T), g.astype(jnp.float32)) + b3)
    s_all = jnp.concatenate(svals, axis=0)
    y = jnp.sum(s_all, axis=1, keepdims=True)
    val = (jnp.transpose(y) + jnp.concatenate(offs, axis=1)) * (1.0 / CLS)
    out_ref[0] = val


def kernel(x_in, W_cl1, b_cl1, g1, be1, W_cl2, b_cl2, W_cl3, b_cl3,
           W_reg1, b_reg1, gr, br, W_cm2, b_cm2, W_cm3, b_cm3):
    B, Cin, H, Wd = x_in.shape
    N = B * H * Wd
    NTILES = N // WT
    NP = N + CLS * T
    NPT = NP // T
    GIDPAD = ((NPT + 1 + LANE - 1) // LANE) * LANE
    grid = (B, (H * Wd) // WT)

    def _col(v):
        return v.reshape(-1, 1)

    Wcl3p = jnp.zeros((CH, CH), jnp.float32).at[: CLS + 1, :].set(W_cl3)
    bcl3p = jnp.zeros((CH,), jnp.float32).at[: CLS + 1].set(b_cl3)

    wpt = Wd // WT
    inv_n = jnp.full((1, 1), 1.0 / N, jnp.float32)
    mask4d, inds4d, xr_tm, hist_t = pl.pallas_call(
        _fused_body,
        grid=(2,) + grid,
        in_specs=[
            pl.BlockSpec((1, Cin, 1, WT), lambda p, b, w: (b, 0, 0, w)),
            pl.BlockSpec((CH, Cin), lambda p, b, w: (0, 0)),
            pl.BlockSpec((CH, 1), lambda p, b, w: (0, 0)),
            pl.BlockSpec((CH, 1), lambda p, b, w: (0, 0)),
            pl.BlockSpec((CH, CH), lambda p, b, w: (0, 0)),
            pl.BlockSpec((CH, 1), lambda p, b, w: (0, 0)),
            pl.BlockSpec((CH, CH), lambda p, b, w: (0, 0)),
            pl.BlockSpec((CH, 1), lambda p, b, w: (0, 0)),
            pl.BlockSpec((CH, Cin), lambda p, b, w: (0, 0)),
            pl.BlockSpec((CH, 1), lambda p, b, w: (0, 0)),
            pl.BlockSpec((CH, 1), lambda p, b, w: (0, 0)),
            pl.BlockSpec((1, 1), lambda p, b, w: (0, 0)),
        ],
        out_specs=[
            pl.BlockSpec((1, 1, 1, WT), lambda p, b, w: (p * b, 0, 0, p * w)),
            pl.BlockSpec((1, 1, 1, WT), lambda p, b, w: (p * b, 0, 0, p * w)),
            pl.BlockSpec((WT, CH),
                         lambda p, b, w, _wpt=wpt: (p * (b * _wpt + w), 0)),
            pl.BlockSpec((1, 1, CLS),
                         lambda p, b, w, _wpt=wpt: (p * (b * _wpt + w), 0, 0)),
        ],
        out_shape=[
            jax.ShapeDtypeStruct((B, 1, H, Wd), jnp.float32),
            jax.ShapeDtypeStruct((B, 1, H, Wd), jnp.int32),
            jax.ShapeDtypeStruct((N, CH), jnp.float32),
            jax.ShapeDtypeStruct((NTILES, 1, CLS), jnp.int32),
        ],
        scratch_shapes=[pltpu.VMEM((CH, 4), jnp.float32)],
    )(x_in, W_cl1, _col(g1), _col(be1), W_cl2, _col(b_cl2), Wcl3p, _col(bcl3p),
      W_reg1, _col(gr), _col(br), inv_n)

    mesh = plsc.VectorSubcoreMesh(core_axis_name="c", subcore_axis_name="s",
                                  num_cores=2, num_subcores=16)
    route = _make_route_kernel(N, NTILES, NP, NPT, GIDPAD, mesh)
    dest, xs, gid = route(inds4d.reshape(N), hist_t.reshape(NTILES, CLS), xr_tm)

    w3p = jnp.transpose(W_cm3, (0, 2, 1))
    b3p = jnp.pad(b_cm3[:, None, :], ((0, 0), (0, 0), (0, 31)))
    wpack = jnp.concatenate(
        [W_cm2, b_cm2[:, None, :], w3p, b3p,
         jnp.zeros((CLS, 5, 32), jnp.float32)], axis=1).astype(jnp.bfloat16)

    def _wmap(k):
        return pl.BlockSpec(
            (1, CH + 8, 32),
            lambda i, gid_ref, _k=k: (gid_ref[i * GB + _k], 0, 0))

    val_sorted = pl.pallas_call(
        _group_body,
        grid_spec=pltpu.PrefetchScalarGridSpec(
            num_scalar_prefetch=1,
            grid=(NPT // GB,),
            in_specs=[pl.BlockSpec((GB * T, CH), lambda i, gid_ref: (i, 0))]
            + [_wmap(k) for k in range(GB)],
            out_specs=pl.BlockSpec((1, 1, GB * T), lambda i, gid_ref: (i, 0, 0)),
        ),
        out_shape=jax.ShapeDtypeStruct((NPT // GB, 1, GB * T), jnp.float32),
    )(gid, xs, *([wpack] * GB))

    gather_back = _make_gather_kernel(N, NP, mesh)
    xreal_flat = gather_back(val_sorted.reshape(NP), dest)

    x_real = xreal_flat.reshape(B, 1, H, Wd)
    return (x_real, mask4d)


# speedup vs baseline: 1.3334x; 1.3334x over previous
import functools
import jax
import jax.numpy as jnp
from jax import lax
from jax.experimental import pallas as pl
from jax.experimental.pallas import tpu as pltpu
from jax.experimental.pallas import tpu_sc as plsc

CLS = 64
CH = 128
WT = 1024
GB = 16
T = 128
NW = 32
LANE = 16


def _lrelu(x):
    return jnp.where(x >= 0, x, 0.01 * x)


def _vgather(vec, idx):
    dn = lax.GatherDimensionNumbers(
        offset_dims=(), collapsed_slice_dims=(0,), start_index_map=(0,))
    return lax.gather(vec, idx[:, None], dn, (1,),
                      mode=lax.GatherScatterMode.PROMISE_IN_BOUNDS)


def _fused_body(
    x_ref, wcl1_ref, g1_ref, be1_ref, wcl2_ref, bcl2_ref, wcl3_ref, bcl3_ref,
    wrg_ref, gr_ref, br_ref, inv_n_ref,
    mask_ref, inds_ref, xr_ref, hist_ref, acc,
):
    p = pl.program_id(0)
    step = pl.program_id(1) * pl.num_programs(2) + pl.program_id(2)
    x = x_ref[0, :, 0, :]
    wt = x.shape[1]

    @pl.when(p == 0)
    def _():
        ycl = jnp.dot(wcl1_ref[...], x, preferred_element_type=jnp.float32)
        yrg = jnp.dot(wrg_ref[...], x, preferred_element_type=jnp.float32)

        def _half(lo):
            yc = ycl[:, lo:lo + 512]
            yr2 = yrg[:, lo:lo + 512]
            return jnp.concatenate(
                [
                    jnp.sum(yc, axis=1, keepdims=True),
                    jnp.sum(yc * yc, axis=1, keepdims=True),
                    jnp.sum(yr2, axis=1, keepdims=True),
                    jnp.sum(yr2 * yr2, axis=1, keepdims=True),
                ],
                axis=1,
            )

        acc[...] = jnp.where(step == 0, _half(0), acc[...] + _half(0)) + _half(512)

    @pl.when(p == 1)
    def _():
        eps = 1e-5
        inv_n = inv_n_ref[0, 0]
        st = acc[...] * inv_n
        m_cl, q_cl = st[:, 0:1], st[:, 1:2]
        m_rg, q_rg = st[:, 2:3], st[:, 3:4]
        sc1 = g1_ref[...] / jnp.sqrt(q_cl - m_cl * m_cl + eps)
        sh1 = be1_ref[...] - m_cl * sc1
        scr = gr_ref[...] / jnp.sqrt(q_rg - m_rg * m_rg + eps)
        shr = br_ref[...] - m_rg * scr

        y = jnp.dot(wcl1_ref[...], x, preferred_element_type=jnp.float32)
        h1 = _lrelu(y * sc1 + sh1)
        h2 = _lrelu(jnp.dot(wcl2_ref[...], h1, preferred_element_type=jnp.float32)
                    + bcl2_ref[...][:, 0:1])
        logits = (jnp.dot(wcl3_ref[...], h2, preferred_element_type=jnp.float32)
                  + bcl3_ref[...][:, 0:1])
        cls = logits[0:CLS, :]
        m = jnp.max(cls, axis=0, keepdims=True)
        iota = lax.broadcasted_iota(jnp.int32, (CLS, wt), 0)
        ind = jnp.min(jnp.where(cls == m, iota, CLS), axis=0, keepdims=True)
        mask_ref[0, 0] = _lrelu(logits[CLS : CLS + 1, :])
        inds_ref[0, 0] = ind

        onehot = (iota == ind).astype(jnp.int32)
        hsum = jnp.sum(onehot, axis=1, keepdims=True)
        hist_ref[0] = jnp.transpose(hsum)

        yr = jnp.dot(wrg_ref[...], x, preferred_element_type=jnp.float32)
        xr = _lrelu(yr * scr + shr)
        xr_ref[...] = jnp.transpose(xr)


def _rank_in_vreg(idx, lane):
    sk, sv = plsc.sort_key_val(idx, lane)
    prev = _vgather(sk, jnp.maximum(lane - 1, 0))
    change = (lane == 0) | (sk != prev)
    start = plsc.cummax(jnp.where(change, lane, 0))
    rank = lane - start
    nxt = _vgather(sk, jnp.minimum(lane + 1, 15))
    is_last = (lane == 15) | (sk != nxt)
    return sk, sv, rank, is_last


def _make_route_kernel(N, NTILES, NP, NPT, GIDPAD, mesh):
    chunk = N // NW
    nv = chunk // LANE

    @functools.partial(
        pl.kernel, mesh=mesh,
        out_type=[
            jax.ShapeDtypeStruct((NW, chunk // T, T), jnp.int32),
            jax.ShapeDtypeStruct((NP, CH), jnp.float32),
            jax.ShapeDtypeStruct((NPT,), jnp.int32),
        ],
        scratch_types=[
            pltpu.VMEM((chunk,), jnp.int32),
            pltpu.VMEM((NTILES, CLS), jnp.int32),
            pltpu.VMEM((CLS,), jnp.int32),
            pltpu.VMEM((chunk // T, T), jnp.int32),
            pltpu.VMEM((LANE,), jnp.int32),
            pltpu.VMEM((GIDPAD,), jnp.int32),
            pltpu.VMEM((2, T * 2, CH), jnp.float32),
            pltpu.SemaphoreType.DMA,
            pltpu.SemaphoreType.DMA,
            pltpu.SemaphoreType.DMA,
            pltpu.SemaphoreType.DMA,
        ],
        compiler_params=pltpu.CompilerParams(needs_layout_passes=False),
    )
    def route(inds_hbm, hist_hbm, xr_hbm, dest_hbm, xs_hbm, gid_hbm,
              idx_v, hist_v, cnt_v, dest2d_v, tmp16_v, gid_v, rows_v,
              semc0, semc1, sems0, sems1):
        wid = lax.axis_index("s") * 2 + lax.axis_index("c")
        lane = lax.iota(jnp.int32, LANE)
        pltpu.sync_copy(hist_hbm, hist_v)
        pltpu.sync_copy(inds_hbm.at[pl.ds(wid * chunk, chunk)], idx_v)

        tiles_per_chunk = NTILES // NW
        tot, par = [], []
        for j in range(4):
            acc = jnp.zeros((LANE,), jnp.int32)
            pacc = jnp.zeros((LANE,), jnp.int32)
            for t in range(NTILES):
                row = hist_v[t, pl.ds(j * LANE, LANE)]
                acc = acc + row
                pred = (t < wid * tiles_per_chunk).astype(jnp.int32)
                pacc = pacc + row * pred
            tot.append(acc)
            par.append(pacc)

        carry = jnp.int32(0)
        tile_off = []
        for j in range(4):
            aligned = ((tot[j] + (T - 1)) >> 7) << 7
            cs = plsc.cumsum(aligned)
            off = cs - aligned + carry
            carry = carry + jnp.sum(aligned)
            cnt_v[pl.ds(j * LANE, LANE)] = off + par[j]
            tile_off.append(off >> 7)

        @pl.when(wid == 0)
        def _():
            for jv in range(GIDPAD // LANE):
                gid_v[pl.ds(jv * LANE, LANE)] = jnp.zeros((LANE,), jnp.int32)
            for j in range(4):
                sk, sv, rank, is_last = _rank_in_vreg(tile_off[j], lane)
                base = plsc.load_gather(gid_v, [sk])
                plsc.store_scatter(gid_v, [sk], base + rank + 1, mask=is_last)
            c2 = jnp.int32(0)
            for jv in range(GIDPAD // LANE):
                seg = gid_v[pl.ds(jv * LANE, LANE)]
                s = jnp.sum(seg)
                gid_v[pl.ds(jv * LANE, LANE)] = plsc.cumsum(seg) + c2 - 1
                c2 = c2 + s
            pltpu.sync_copy(gid_v.at[pl.ds(0, NPT)], gid_hbm)

        for v in range(nv):
            idx = idx_v[pl.ds(v * LANE, LANE)]
            sk, sv, rank, is_last = _rank_in_vreg(idx, lane)
            base = plsc.load_gather(cnt_v, [sk])
            plsc.store_scatter(cnt_v, [sk], base + rank + 1, mask=is_last)
            plsc.store_scatter(tmp16_v, [sv], base + rank)
            dest2d_v[v // 8, pl.ds((v % 8) * LANE, LANE)] = tmp16_v[...]

        pltpu.sync_copy(dest2d_v, dest_hbm.at[wid])

        nsub = chunk // (T * 2)
        semc = [semc0, semc1]
        sems = [sems0, sems1]

        def _start_copy(sub):
            buf = sub % 2
            return pltpu.async_copy(
                xr_hbm.at[pl.ds(wid * chunk + sub * T * 2, T * 2)],
                rows_v.at[buf], semc[buf])

        copy_descs = {0: _start_copy(0)}
        scat_descs = {}
        for sub in range(nsub):
            buf = sub % 2
            copy_descs[sub].wait()
            scat_descs[sub] = [
                pltpu.async_copy(
                    rows_v.at[buf, pl.ds(j * T, T)],
                    xs_hbm.at[dest2d_v.at[sub * 2 + j]],
                    sems[buf],
                )
                for j in range(2)
            ]
            if sub + 1 < nsub:
                if sub >= 1:
                    for d in scat_descs[sub - 1]:
                        d.wait()
                copy_descs[sub + 1] = _start_copy(sub + 1)
        for sub in (nsub - 2, nsub - 1):
            if sub >= 0 and sub in scat_descs:
                for d in scat_descs[sub]:
                    d.wait()

    return route


def _make_gather_kernel(N, NP, mesh):
    chunk = N // NW
    nv = chunk // LANE

    @functools.partial(
        pl.kernel, mesh=mesh,
        out_type=jax.ShapeDtypeStruct((NW, chunk // T, T), jnp.float32),
        scratch_types=[
            pltpu.VMEM((NP,), jnp.float32),
            pltpu.VMEM((chunk // T, T), jnp.int32),
            pltpu.VMEM((chunk // T, T), jnp.float32),
        ],
        compiler_params=pltpu.CompilerParams(needs_layout_passes=False),
    )
    def gather_back(val_hbm, dest_hbm, out_hbm, val_v, dv, ov):
        wid = lax.axis_index("s") * 2 + lax.axis_index("c")
        pltpu.sync_copy(val_hbm, val_v)
        pltpu.sync_copy(dest_hbm.at[wid], dv)
        for v in range(nv):
            d = dv[v // 8, pl.ds((v % 8) * LANE, LANE)]
            ov[v // 8, pl.ds((v % 8) * LANE, LANE)] = plsc.load_gather(val_v, [d])
        pltpu.sync_copy(ov, out_hbm.at[wid])

    return gather_back


def _group_body(gid_ref, xs_ref, *refs):
    pk_refs = refs[0:GB]
    out_ref = refs[GB]
    i = pl.program_id(0)
    svals = []
    offs = []
    for k in range(GB):
        g = gid_ref[i * GB + k]
        x = xs_ref[pl.ds(k * T, T), :].astype(jnp.bfloat16)
        pk = pk_refs[k][0]
        z = jnp.dot(x, pk[0:CH, :], preferred_element_type=jnp.float32)
        b2 = pk[CH:CH + 1, :].astype(jnp.float32)
        w3 = pk[CH + 1:CH + 2, :].astype(jnp.float32)
        b3 = pk[CH + 2:CH + 3, 0:1].astype(jnp.float32)
        svals.append(_lrelu(z + b2) * w3)
        offs.append(jnp.full((1, T), g.astype(jnp.float32)) + b3)
    s_all = jnp.concatenate(svals, axis=0)
    y = jnp.sum(s_all, axis=1, keepdims=True)
    val = (jnp.transpose(y) + jnp.concatenate(offs, axis=1)) * (1.0 / CLS)
    out_ref[0] = val


def kernel(x_in, W_cl1, b_cl1, g1, be1, W_cl2, b_cl2, W_cl3, b_cl3,
           W_reg1, b_reg1, gr, br, W_cm2, b_cm2, W_cm3, b_cm3):
    B, Cin, H, Wd = x_in.shape
    N = B * H * Wd
    NTILES = N // WT
    NP = N + CLS * T
    NPT = NP // T
    GIDPAD = ((NPT + 1 + LANE - 1) // LANE) * LANE
    grid = (B, (H * Wd) // WT)

    def _col(v):
        return v.reshape(-1, 1)

    Wcl3p = jnp.zeros((CH, CH), jnp.float32).at[: CLS + 1, :].set(W_cl3)
    bcl3p = jnp.zeros((CH,), jnp.float32).at[: CLS + 1].set(b_cl3)

    wpt = Wd // WT
    inv_n = jnp.full((1, 1), 1.0 / N, jnp.float32)
    mask4d, inds4d, xr_tm, hist_t = pl.pallas_call(
        _fused_body,
        grid=(2,) + grid,
        in_specs=[
            pl.BlockSpec((1, Cin, 1, WT), lambda p, b, w: (b, 0, 0, w)),
            pl.BlockSpec((CH, Cin), lambda p, b, w: (0, 0)),
            pl.BlockSpec((CH, 1), lambda p, b, w: (0, 0)),
            pl.BlockSpec((CH, 1), lambda p, b, w: (0, 0)),
            pl.BlockSpec((CH, CH), lambda p, b, w: (0, 0)),
            pl.BlockSpec((CH, 1), lambda p, b, w: (0, 0)),
            pl.BlockSpec((CH, CH), lambda p, b, w: (0, 0)),
            pl.BlockSpec((CH, 1), lambda p, b, w: (0, 0)),
            pl.BlockSpec((CH, Cin), lambda p, b, w: (0, 0)),
            pl.BlockSpec((CH, 1), lambda p, b, w: (0, 0)),
            pl.BlockSpec((CH, 1), lambda p, b, w: (0, 0)),
            pl.BlockSpec((1, 1), lambda p, b, w: (0, 0)),
        ],
        out_specs=[
            pl.BlockSpec((1, 1, 1, WT), lambda p, b, w: (p * b, 0, 0, p * w)),
            pl.BlockSpec((1, 1, 1, WT), lambda p, b, w: (p * b, 0, 0, p * w)),
            pl.BlockSpec((WT, CH),
                         lambda p, b, w, _wpt=wpt: (p * (b * _wpt + w), 0)),
            pl.BlockSpec((1, 1, CLS),
                         lambda p, b, w, _wpt=wpt: (p * (b * _wpt + w), 0, 0)),
        ],
        out_shape=[
            jax.ShapeDtypeStruct((B, 1, H, Wd), jnp.float32),
            jax.ShapeDtypeStruct((B, 1, H, Wd), jnp.int32),
            jax.ShapeDtypeStruct((N, CH), jnp.float32),
            jax.ShapeDtypeStruct((NTILES, 1, CLS), jnp.int32),
        ],
        scratch_shapes=[pltpu.VMEM((CH, 4), jnp.float32)],
    )(x_in, W_cl1, _col(g1), _col(be1), W_cl2, _col(b_cl2), Wcl3p, _col(bcl3p),
      W_reg1, _col(gr), _col(br), inv_n)

    mesh = plsc.VectorSubcoreMesh(core_axis_name="c", subcore_axis_name="s",
                                  num_cores=2, num_subcores=16)
    route = _make_route_kernel(N, NTILES, NP, NPT, GIDPAD, mesh)
    dest, xs, gid = route(inds4d.reshape(N), hist_t.reshape(NTILES, CLS), xr_tm)

    w3p = jnp.transpose(W_cm3, (0, 2, 1))
    b3p = jnp.pad(b_cm3[:, None, :], ((0, 0), (0, 0), (0, 31)))
    wpack = jnp.concatenate(
        [W_cm2, b_cm2[:, None, :], w3p, b3p,
         jnp.zeros((CLS, 5, 32), jnp.float32)], axis=1).astype(jnp.bfloat16)

    def _wmap(k):
        return pl.BlockSpec(
            (1, CH + 8, 32),
            lambda i, gid_ref, _k=k: (gid_ref[i * GB + _k], 0, 0))

    val_sorted = pl.pallas_call(
        _group_body,
        grid_spec=pltpu.PrefetchScalarGridSpec(
            num_scalar_prefetch=1,
            grid=(NPT // GB,),
            in_specs=[pl.BlockSpec((GB * T, CH), lambda i, gid_ref: (i, 0))]
            + [_wmap(k) for k in range(GB)],
            out_specs=pl.BlockSpec((1, 1, GB * T), lambda i, gid_ref: (i, 0, 0)),
        ),
        out_shape=jax.ShapeDtypeStruct((NPT // GB, 1, GB * T), jnp.float32),
    )(gid, xs, *([wpack] * GB))

    gather_back = _make_gather_kernel(N, NP, mesh)
    xreal_flat = gather_back(val_sorted.reshape(NP), dest)

    x_real = xreal_flat.reshape(B, 1, H, Wd)
    return (x_real, mask4d)
